# Initial kernel scaffold; baseline (speedup 1.0000x reference)
import jax
import jax.numpy as jnp
from jax.experimental import pallas as pl


def kernel(x, edge_index, W, b):
    raise NotImplementedError("write your pallas kernel here")


# speedup vs baseline: 24.6024x; 24.6024x over previous
import functools

import jax
import jax.numpy as jnp
from jax import lax
from jax.experimental import pallas as pl
from jax.experimental.pallas import tpu as pltpu
from jax.experimental.pallas import tpu_sc as plsc

N = 10000
D = 128
C = 64
E = 320000

NC = 2
NS = 16
NW = NC * NS
EPW = E // NW
CH = 125
NCH = EPW // CH
NP = 10240
RPT = NP // NS
DW = 16

BT = 1000
GT = N // BT

_mesh = plsc.VectorSubcoreMesh(core_axis_name="c", subcore_axis_name="s")
_sc_params = pltpu.CompilerParams(use_tc_tiling_on_sc=False)


@functools.partial(
    pl.kernel,
    mesh=_mesh,
    compiler_params=_sc_params,
    out_type=[jax.ShapeDtypeStruct((NP, DW), jnp.float32)] * 2,
    scratch_types=[
        pltpu.VMEM((NCH, CH), jnp.int32),
        pltpu.VMEM((CH, DW), jnp.float32),
        pltpu.VMEM_SHARED((NP, DW), jnp.float32),
    ],
)
def _sc_deg(col_hbm, ones_hbm, zero_hbm, out0, out1, colall, onesbuf, dacc):
    cid = lax.axis_index("c")
    sid = lax.axis_index("s")
    wid = sid * NC + cid
    pltpu.sync_copy(zero_hbm, dacc.at[pl.ds(sid * RPT, RPT)])
    pltpu.sync_copy(ones_hbm, onesbuf)
    pltpu.sync_copy(col_hbm.at[pl.ds(wid * NCH, NCH)], colall)
    plsc.subcore_barrier()

    def body(step, carry):
        pltpu.sync_copy(onesbuf, dacc.at[colall.at[step]], add=True)
        return carry

    lax.fori_loop(0, NCH, body, 0)
    plsc.subcore_barrier()

    @pl.when(cid == 0)
    def _():
        pltpu.sync_copy(dacc.at[pl.ds(sid * RPT, RPT)],
                        out0.at[pl.ds(sid * RPT, RPT)])

    @pl.when(cid == 1)
    def _():
        pltpu.sync_copy(dacc.at[pl.ds(sid * RPT, RPT)],
                        out1.at[pl.ds(sid * RPT, RPT)])


@functools.partial(
    pl.kernel,
    mesh=_mesh,
    compiler_params=_sc_params,
    out_type=[jax.ShapeDtypeStruct((NP, C), jnp.float32)] * 2,
    scratch_types=[
        pltpu.VMEM((NCH, CH), jnp.int32),
        pltpu.VMEM((NCH, CH), jnp.int32),
        pltpu.VMEM((CH, C), jnp.float32),
        pltpu.VMEM_SHARED((NP, C), jnp.float32),
        pltpu.SemaphoreType.DMA,
    ],
)
def _sc_hop(u_hbm, row_hbm, col_hbm, zero_hbm, out0, out1,
            rowall, colall, rows, acc, sem):
    cid = lax.axis_index("c")
    sid = lax.axis_index("s")
    wid = sid * NC + cid
    pltpu.sync_copy(zero_hbm, acc.at[pl.ds(sid * RPT, RPT)])
    pltpu.sync_copy(row_hbm.at[pl.ds(wid * NCH, NCH)], rowall)
    pltpu.sync_copy(col_hbm.at[pl.ds(wid * NCH, NCH)], colall)
    plsc.subcore_barrier()

    def body(step, carry):
        pltpu.async_copy(u_hbm.at[rowall.at[step]], rows, sem).wait()
        pltpu.sync_copy(rows, acc.at[colall.at[step]], add=True)
        return carry

    lax.fori_loop(0, NCH, body, 0)
    plsc.subcore_barrier()

    @pl.when(cid == 0)
    def _():
        pltpu.sync_copy(acc.at[pl.ds(sid * RPT, RPT)],
                        out0.at[pl.ds(sid * RPT, RPT)])

    @pl.when(cid == 1)
    def _():
        pltpu.sync_copy(acc.at[pl.ds(sid * RPT, RPT)],
                        out1.at[pl.ds(sid * RPT, RPT)])


def _tc_prep_body(x_ref, w_ref, d0_ref, d1_ref, z_ref, u0_ref):
    z = lax.dot_general(x_ref[...], w_ref[...], (((1,), (1,)), ((), ())),
                        preferred_element_type=jnp.float32)
    deg = d0_ref[...][:, :1] + d1_ref[...][:, :1] + 2.0
    z_ref[...] = z
    u0_ref[...] = z * lax.rsqrt(deg)


_tc_prep = pl.pallas_call(
    _tc_prep_body,
    grid=(GT,),
    in_specs=[
        pl.BlockSpec((BT, D), lambda i: (i, 0)),
        pl.BlockSpec((C, D), lambda i: (0, 0)),
        pl.BlockSpec((BT, DW), lambda i: (i, 0)),
        pl.BlockSpec((BT, DW), lambda i: (i, 0)),
    ],
    out_specs=[pl.BlockSpec((BT, C), lambda i: (i, 0))] * 2,
    out_shape=[jax.ShapeDtypeStruct((N, C), jnp.float32)] * 2,
)


def _tc_comb1_body(p0_ref, p1_ref, d0_ref, d1_ref, h_ref, h1_ref, u1_ref):
    deg = d0_ref[...][:, :1] + d1_ref[...][:, :1] + 2.0
    dis = lax.rsqrt(deg)
    h1 = dis * (p0_ref[...] + p1_ref[...]) + (2.0 / deg) * h_ref[...]
    h1_ref[...] = h1
    u1_ref[...] = dis * h1


_tc_comb1 = pl.pallas_call(
    _tc_comb1_body,
    grid=(GT,),
    in_specs=[
        pl.BlockSpec((BT, C), lambda i: (i, 0)),
        pl.BlockSpec((BT, C), lambda i: (i, 0)),
        pl.BlockSpec((BT, DW), lambda i: (i, 0)),
        pl.BlockSpec((BT, DW), lambda i: (i, 0)),
        pl.BlockSpec((BT, C), lambda i: (i, 0)),
    ],
    out_specs=[pl.BlockSpec((BT, C), lambda i: (i, 0))] * 2,
    out_shape=[jax.ShapeDtypeStruct((N, C), jnp.float32)] * 2,
)


def _tc_comb2_body(p0_ref, p1_ref, d0_ref, d1_ref, h_ref, b_ref, o_ref):
    deg = d0_ref[...][:, :1] + d1_ref[...][:, :1] + 2.0
    dis = lax.rsqrt(deg)
    t = dis * (p0_ref[...] + p1_ref[...]) + (2.0 / deg) * h_ref[...] + b_ref[...]
    m = jnp.max(t, axis=1, keepdims=True)
    lse = jnp.log(jnp.sum(jnp.exp(t - m), axis=1, keepdims=True)) + m
    o_ref[...] = t - lse


_tc_comb2 = pl.pallas_call(
    _tc_comb2_body,
    grid=(GT,),
    in_specs=[
        pl.BlockSpec((BT, C), lambda i: (i, 0)),
        pl.BlockSpec((BT, C), lambda i: (i, 0)),
        pl.BlockSpec((BT, DW), lambda i: (i, 0)),
        pl.BlockSpec((BT, DW), lambda i: (i, 0)),
        pl.BlockSpec((BT, C), lambda i: (i, 0)),
        pl.BlockSpec((1, C), lambda i: (0, 0)),
    ],
    out_specs=pl.BlockSpec((BT, C), lambda i: (i, 0)),
    out_shape=jax.ShapeDtypeStruct((N, C), jnp.float32),
)


def kernel(x, edge_index, W, b):
    row = edge_index[0].reshape(NW * NCH, CH)
    col = edge_index[1].reshape(NW * NCH, CH)
    ones_d = jnp.ones((CH, DW), jnp.float32)
    zero_d = jnp.zeros((RPT, DW), jnp.float32)
    zero_c = jnp.zeros((RPT, C), jnp.float32)

    dg0, dg1 = _sc_deg(col, ones_d, zero_d)
    z, u0 = _tc_prep(x, W, dg0, dg1)
    p10, p11 = _sc_hop(u0, row, col, zero_c)
    h1, u1 = _tc_comb1(p10, p11, dg0, dg1, z)
    p20, p21 = _sc_hop(u1, row, col, zero_c)
    return _tc_comb2(p20, p21, dg0, dg1, h1, b.reshape(1, C))


# speedup vs baseline: 33.3657x; 1.3562x over previous
import functools

import jax
import jax.numpy as jnp
from jax import lax
from jax.experimental import pallas as pl
from jax.experimental.pallas import tpu as pltpu
from jax.experimental.pallas import tpu_sc as plsc

N = 10000
D = 128
C = 64
E = 320000

NC = 2
NS = 16
NW = NC * NS
EPW = E // NW
CH = 125
NCH = EPW // CH
NP = 10240
RPT = NP // NS
DW = 16

BT = 1000
GT = N // BT

_mesh = plsc.VectorSubcoreMesh(core_axis_name="c", subcore_axis_name="s")
_sc_params = pltpu.CompilerParams(use_tc_tiling_on_sc=False)


@functools.partial(
    pl.kernel,
    mesh=_mesh,
    compiler_params=_sc_params,
    out_type=[jax.ShapeDtypeStruct((NP, DW), jnp.float32)] * 2,
    scratch_types=[
        pltpu.VMEM((NCH, CH), jnp.int32),
        pltpu.VMEM((CH, DW), jnp.float32),
        pltpu.VMEM_SHARED((NP, DW), jnp.float32),
    ],
)
def _sc_deg(col_hbm, ones_hbm, zero_hbm, out0, out1, colall, onesbuf, dacc):
    cid = lax.axis_index("c")
    sid = lax.axis_index("s")
    wid = sid * NC + cid
    pltpu.sync_copy(zero_hbm, dacc.at[pl.ds(sid * RPT, RPT)])
    pltpu.sync_copy(ones_hbm, onesbuf)
    pltpu.sync_copy(col_hbm.at[pl.ds(wid * NCH, NCH)], colall)
    plsc.subcore_barrier()

    def body(step, carry):
        pltpu.sync_copy(onesbuf, dacc.at[colall.at[step]], add=True)
        return carry

    lax.fori_loop(0, NCH, body, 0)
    plsc.subcore_barrier()

    @pl.when(cid == 0)
    def _():
        pltpu.sync_copy(dacc.at[pl.ds(sid * RPT, RPT)],
                        out0.at[pl.ds(sid * RPT, RPT)])

    @pl.when(cid == 1)
    def _():
        pltpu.sync_copy(dacc.at[pl.ds(sid * RPT, RPT)],
                        out1.at[pl.ds(sid * RPT, RPT)])


@functools.partial(
    pl.kernel,
    mesh=_mesh,
    compiler_params=_sc_params,
    out_type=[jax.ShapeDtypeStruct((NP, C), jnp.float32)] * 2,
    scratch_types=[
        pltpu.VMEM((NCH, CH), jnp.int32),
        pltpu.VMEM((NCH, CH), jnp.int32),
        pltpu.VMEM((2, CH, C), jnp.float32),
        pltpu.VMEM_SHARED((NP, C), jnp.float32),
        pltpu.SemaphoreType.DMA,
        pltpu.SemaphoreType.DMA,
    ],
)
def _sc_hop(u_hbm, row_hbm, col_hbm, zero_hbm, out0, out1,
            rowall, colall, rows, acc, sem0, sem1):
    cid = lax.axis_index("c")
    sid = lax.axis_index("s")
    wid = sid * NC + cid
    pltpu.sync_copy(zero_hbm, acc.at[pl.ds(sid * RPT, RPT)])
    pltpu.sync_copy(row_hbm.at[pl.ds(wid * NCH, NCH)], rowall)
    pltpu.sync_copy(col_hbm.at[pl.ds(wid * NCH, NCH)], colall)
    plsc.subcore_barrier()

    pltpu.async_copy(u_hbm.at[rowall.at[0]], rows.at[0], sem0)
    pltpu.async_copy(u_hbm.at[rowall.at[1]], rows.at[1], sem1)

    def body(g, carry):
        s0 = g * 2
        s1 = s0 + 1
        pltpu.make_async_copy(u_hbm.at[rowall.at[s0]], rows.at[0], sem0).wait()
        pltpu.sync_copy(rows.at[0], acc.at[colall.at[s0]], add=True)

        @pl.when(s0 + 2 < NCH)
        def _():
            pltpu.async_copy(u_hbm.at[rowall.at[s0 + 2]], rows.at[0], sem0)

        pltpu.make_async_copy(u_hbm.at[rowall.at[s1]], rows.at[1], sem1).wait()
        pltpu.sync_copy(rows.at[1], acc.at[colall.at[s1]], add=True)

        @pl.when(s1 + 2 < NCH)
        def _():
            pltpu.async_copy(u_hbm.at[rowall.at[s1 + 2]], rows.at[1], sem1)

        return carry

    lax.fori_loop(0, NCH // 2, body, 0)
    plsc.subcore_barrier()

    @pl.when(cid == 0)
    def _():
        pltpu.sync_copy(acc.at[pl.ds(sid * RPT, RPT)],
                        out0.at[pl.ds(sid * RPT, RPT)])

    @pl.when(cid == 1)
    def _():
        pltpu.sync_copy(acc.at[pl.ds(sid * RPT, RPT)],
                        out1.at[pl.ds(sid * RPT, RPT)])


def _tc_prep_body(x_ref, w_ref, d0_ref, d1_ref, z_ref, u0_ref):
    z = lax.dot_general(x_ref[...], w_ref[...], (((1,), (1,)), ((), ())),
                        preferred_element_type=jnp.float32)
    deg = d0_ref[...][:, :1] + d1_ref[...][:, :1] + 2.0
    z_ref[...] = z
    u0_ref[...] = z * lax.rsqrt(deg)


_tc_prep = pl.pallas_call(
    _tc_prep_body,
    grid=(GT,),
    in_specs=[
        pl.BlockSpec((BT, D), lambda i: (i, 0)),
        pl.BlockSpec((C, D), lambda i: (0, 0)),
        pl.BlockSpec((BT, DW), lambda i: (i, 0)),
        pl.BlockSpec((BT, DW), lambda i: (i, 0)),
    ],
    out_specs=[pl.BlockSpec((BT, C), lambda i: (i, 0))] * 2,
    out_shape=[jax.ShapeDtypeStruct((N, C), jnp.float32)] * 2,
)


def _tc_comb1_body(p0_ref, p1_ref, d0_ref, d1_ref, h_ref, h1_ref, u1_ref):
    deg = d0_ref[...][:, :1] + d1_ref[...][:, :1] + 2.0
    dis = lax.rsqrt(deg)
    h1 = dis * (p0_ref[...] + p1_ref[...]) + (2.0 / deg) * h_ref[...]
    h1_ref[...] = h1
    u1_ref[...] = dis * h1


_tc_comb1 = pl.pallas_call(
    _tc_comb1_body,
    grid=(GT,),
    in_specs=[
        pl.BlockSpec((BT, C), lambda i: (i, 0)),
        pl.BlockSpec((BT, C), lambda i: (i, 0)),
        pl.BlockSpec((BT, DW), lambda i: (i, 0)),
        pl.BlockSpec((BT, DW), lambda i: (i, 0)),
        pl.BlockSpec((BT, C), lambda i: (i, 0)),
    ],
    out_specs=[pl.BlockSpec((BT, C), lambda i: (i, 0))] * 2,
    out_shape=[jax.ShapeDtypeStruct((N, C), jnp.float32)] * 2,
)


def _tc_comb2_body(p0_ref, p1_ref, d0_ref, d1_ref, h_ref, b_ref, o_ref):
    deg = d0_ref[...][:, :1] + d1_ref[...][:, :1] + 2.0
    dis = lax.rsqrt(deg)
    t = dis * (p0_ref[...] + p1_ref[...]) + (2.0 / deg) * h_ref[...] + b_ref[...]
    m = jnp.max(t, axis=1, keepdims=True)
    lse = jnp.log(jnp.sum(jnp.exp(t - m), axis=1, keepdims=True)) + m
    o_ref[...] = t - lse


_tc_comb2 = pl.pallas_call(
    _tc_comb2_body,
    grid=(GT,),
    in_specs=[
        pl.BlockSpec((BT, C), lambda i: (i, 0)),
        pl.BlockSpec((BT, C), lambda i: (i, 0)),
        pl.BlockSpec((BT, DW), lambda i: (i, 0)),
        pl.BlockSpec((BT, DW), lambda i: (i, 0)),
        pl.BlockSpec((BT, C), lambda i: (i, 0)),
        pl.BlockSpec((1, C), lambda i: (0, 0)),
    ],
    out_specs=pl.BlockSpec((BT, C), lambda i: (i, 0)),
    out_shape=jax.ShapeDtypeStruct((N, C), jnp.float32),
)


def kernel(x, edge_index, W, b):
    row = edge_index[0].reshape(NW * NCH, CH)
    col = edge_index[1].reshape(NW * NCH, CH)
    ones_d = jnp.ones((CH, DW), jnp.float32)
    zero_d = jnp.zeros((RPT, DW), jnp.float32)
    zero_c = jnp.zeros((RPT, C), jnp.float32)

    dg0, dg1 = _sc_deg(col, ones_d, zero_d)
    z, u0 = _tc_prep(x, W, dg0, dg1)
    p10, p11 = _sc_hop(u0, row, col, zero_c)
    h1, u1 = _tc_comb1(p10, p11, dg0, dg1, z)
    p20, p21 = _sc_hop(u1, row, col, zero_c)
    return _tc_comb2(p20, p21, dg0, dg1, h1, b.reshape(1, C))


# speedup vs baseline: 34.3129x; 1.0284x over previous
import functools

import jax
import jax.numpy as jnp
from jax import lax
from jax.experimental import pallas as pl
from jax.experimental.pallas import tpu as pltpu
from jax.experimental.pallas import tpu_sc as plsc

N = 10000
D = 128
C = 64
E = 320000

NC = 2
NS = 16
NW = NC * NS
EPW = E // NW
CH = 125
NCH = EPW // CH
NP = 10240
RPT = NP // NS
DW = 16

BT = 1000
GT = N // BT

_mesh = plsc.VectorSubcoreMesh(core_axis_name="c", subcore_axis_name="s")
_sc_params = pltpu.CompilerParams(use_tc_tiling_on_sc=False)


@functools.partial(
    pl.kernel,
    mesh=_mesh,
    compiler_params=_sc_params,
    out_type=[jax.ShapeDtypeStruct((NP, DW), jnp.float32)] * 2,
    scratch_types=[
        pltpu.VMEM((NCH, CH), jnp.int32),
        pltpu.VMEM((CH, DW), jnp.float32),
        pltpu.VMEM_SHARED((NP, DW), jnp.float32),
    ],
)
def _sc_deg(col_hbm, ones_hbm, zero_hbm, out0, out1, colall, onesbuf, dacc):
    cid = lax.axis_index("c")
    sid = lax.axis_index("s")
    wid = sid * NC + cid
    pltpu.sync_copy(zero_hbm, dacc.at[pl.ds(sid * RPT, RPT)])
    pltpu.sync_copy(ones_hbm, onesbuf)
    pltpu.sync_copy(col_hbm.at[pl.ds(wid * NCH, NCH)], colall)
    plsc.subcore_barrier()

    def body(step, carry):
        pltpu.sync_copy(onesbuf, dacc.at[colall.at[step]], add=True)
        return carry

    lax.fori_loop(0, NCH, body, 0)
    plsc.subcore_barrier()

    @pl.when(cid == 0)
    def _():
        pltpu.sync_copy(dacc.at[pl.ds(sid * RPT, RPT)],
                        out0.at[pl.ds(sid * RPT, RPT)])

    @pl.when(cid == 1)
    def _():
        pltpu.sync_copy(dacc.at[pl.ds(sid * RPT, RPT)],
                        out1.at[pl.ds(sid * RPT, RPT)])


@functools.partial(
    pl.kernel,
    mesh=_mesh,
    compiler_params=_sc_params,
    out_type=[jax.ShapeDtypeStruct((NP, C), jnp.float32)] * 2,
    scratch_types=[
        pltpu.VMEM((NCH, CH), jnp.int32),
        pltpu.VMEM((NCH, CH), jnp.int32),
        pltpu.VMEM((4, CH, C), jnp.float32),
        pltpu.VMEM_SHARED((NP, C), jnp.float32),
        [pltpu.SemaphoreType.DMA] * 4,
        [pltpu.SemaphoreType.DMA] * 4,
    ],
)
def _sc_hop(u_hbm, row_hbm, col_hbm, zero_hbm, out0, out1,
            rowall, colall, rows, acc, semg, sems):
    cid = lax.axis_index("c")
    sid = lax.axis_index("s")
    wid = sid * NC + cid
    pltpu.sync_copy(zero_hbm, acc.at[pl.ds(sid * RPT, RPT)])
    pltpu.sync_copy(row_hbm.at[pl.ds(wid * NCH, NCH)], rowall)
    pltpu.sync_copy(col_hbm.at[pl.ds(wid * NCH, NCH)], colall)
    plsc.subcore_barrier()

    def _gather(s, slot):
        pltpu.async_copy(u_hbm.at[rowall.at[s]], rows.at[slot], semg[slot])

    def _wait_gather(s, slot):
        pltpu.make_async_copy(u_hbm.at[rowall.at[s]], rows.at[slot],
                              semg[slot]).wait()

    def _scatter(s, slot):
        pltpu.async_copy(rows.at[slot], acc.at[colall.at[s]], sems[slot],
                         add=True)

    def _wait_scatter(s, slot):
        pltpu.make_async_copy(rows.at[slot], acc.at[colall.at[s]],
                              sems[slot]).wait()

    _gather(0, 0)
    _gather(1, 1)

    def body(g, carry):
        for j in range(4):
            s = g * 4 + j
            _wait_gather(s, j)
            _scatter(s, j)

            @pl.when(s >= 2)
            def _():
                _wait_scatter(s - 2, (j - 2) % 4)

            @pl.when(s + 2 < NCH)
            def _():
                _gather(s + 2, (j + 2) % 4)

        return carry

    lax.fori_loop(0, NCH // 4, body, 0)
    _wait_scatter(NCH - 2, 2)
    _wait_scatter(NCH - 1, 3)
    plsc.subcore_barrier()

    @pl.when(cid == 0)
    def _():
        pltpu.sync_copy(acc.at[pl.ds(sid * RPT, RPT)],
                        out0.at[pl.ds(sid * RPT, RPT)])

    @pl.when(cid == 1)
    def _():
        pltpu.sync_copy(acc.at[pl.ds(sid * RPT, RPT)],
                        out1.at[pl.ds(sid * RPT, RPT)])


def _tc_prep_body(x_ref, w_ref, d0_ref, d1_ref, z_ref, u0_ref):
    z = lax.dot_general(x_ref[...], w_ref[...], (((1,), (1,)), ((), ())),
                        preferred_element_type=jnp.float32)
    deg = d0_ref[...][:, :1] + d1_ref[...][:, :1] + 2.0
    z_ref[...] = z
    u0_ref[...] = z * lax.rsqrt(deg)


_tc_prep = pl.pallas_call(
    _tc_prep_body,
    grid=(GT,),
    in_specs=[
        pl.BlockSpec((BT, D), lambda i: (i, 0)),
        pl.BlockSpec((C, D), lambda i: (0, 0)),
        pl.BlockSpec((BT, DW), lambda i: (i, 0)),
        pl.BlockSpec((BT, DW), lambda i: (i, 0)),
    ],
    out_specs=[pl.BlockSpec((BT, C), lambda i: (i, 0))] * 2,
    out_shape=[jax.ShapeDtypeStruct((N, C), jnp.float32)] * 2,
)


def _tc_comb1_body(p0_ref, p1_ref, d0_ref, d1_ref, h_ref, h1_ref, u1_ref):
    deg = d0_ref[...][:, :1] + d1_ref[...][:, :1] + 2.0
    dis = lax.rsqrt(deg)
    h1 = dis * (p0_ref[...] + p1_ref[...]) + (2.0 / deg) * h_ref[...]
    h1_ref[...] = h1
    u1_ref[...] = dis * h1


_tc_comb1 = pl.pallas_call(
    _tc_comb1_body,
    grid=(GT,),
    in_specs=[
        pl.BlockSpec((BT, C), lambda i: (i, 0)),
        pl.BlockSpec((BT, C), lambda i: (i, 0)),
        pl.BlockSpec((BT, DW), lambda i: (i, 0)),
        pl.BlockSpec((BT, DW), lambda i: (i, 0)),
        pl.BlockSpec((BT, C), lambda i: (i, 0)),
    ],
    out_specs=[pl.BlockSpec((BT, C), lambda i: (i, 0))] * 2,
    out_shape=[jax.ShapeDtypeStruct((N, C), jnp.float32)] * 2,
)


def _tc_comb2_body(p0_ref, p1_ref, d0_ref, d1_ref, h_ref, b_ref, o_ref):
    deg = d0_ref[...][:, :1] + d1_ref[...][:, :1] + 2.0
    dis = lax.rsqrt(deg)
    t = dis * (p0_ref[...] + p1_ref[...]) + (2.0 / deg) * h_ref[...] + b_ref[...]
    m = jnp.max(t, axis=1, keepdims=True)
    lse = jnp.log(jnp.sum(jnp.exp(t - m), axis=1, keepdims=True)) + m
    o_ref[...] = t - lse


_tc_comb2 = pl.pallas_call(
    _tc_comb2_body,
    grid=(GT,),
    in_specs=[
        pl.BlockSpec((BT, C), lambda i: (i, 0)),
        pl.BlockSpec((BT, C), lambda i: (i, 0)),
        pl.BlockSpec((BT, DW), lambda i: (i, 0)),
        pl.BlockSpec((BT, DW), lambda i: (i, 0)),
        pl.BlockSpec((BT, C), lambda i: (i, 0)),
        pl.BlockSpec((1, C), lambda i: (0, 0)),
    ],
    out_specs=pl.BlockSpec((BT, C), lambda i: (i, 0)),
    out_shape=jax.ShapeDtypeStruct((N, C), jnp.float32),
)


def kernel(x, edge_index, W, b):
    row = edge_index[0].reshape(NW * NCH, CH)
    col = edge_index[1].reshape(NW * NCH, CH)
    ones_d = jnp.ones((CH, DW), jnp.float32)
    zero_d = jnp.zeros((RPT, DW), jnp.float32)
    zero_c = jnp.zeros((RPT, C), jnp.float32)

    dg0, dg1 = _sc_deg(col, ones_d, zero_d)
    z, u0 = _tc_prep(x, W, dg0, dg1)
    p10, p11 = _sc_hop(u0, row, col, zero_c)
    h1, u1 = _tc_comb1(p10, p11, dg0, dg1, z)
    p20, p21 = _sc_hop(u1, row, col, zero_c)
    return _tc_comb2(p20, p21, dg0, dg1, h1, b.reshape(1, C))


# speedup vs baseline: 37.0897x; 1.0809x over previous
import functools

import jax
import jax.numpy as jnp
from jax import lax
from jax.experimental import pallas as pl
from jax.experimental.pallas import tpu as pltpu
from jax.experimental.pallas import tpu_sc as plsc

N = 10000
D = 128
C = 64
E = 320000

NC = 2
NS = 16
NW = NC * NS
EPW = E // NW
CH = 125
NCH = EPW // CH
CBASE = NW * NCH
NP = 10240
RPT = NP // NS
DW = 16
PCH = 128

BT = 1000
GT = N // BT

_mesh = plsc.VectorSubcoreMesh(core_axis_name="c", subcore_axis_name="s")
_sc_params = pltpu.CompilerParams(use_tc_tiling_on_sc=False)


def _memset_zero(buf, nrows, ncols):
    zv = jnp.zeros((16,), jnp.float32)

    def body(r, carry):
        for j in range(ncols // 16):
            buf[r, pl.ds(j * 16, 16)] = zv
        return carry

    lax.fori_loop(0, nrows, body, 0)


def _zero_acc(acc, zbuf, sid, width):
    _memset_zero(zbuf, PCH, width)
    for k in range(RPT // PCH):
        pltpu.sync_copy(zbuf, acc.at[pl.ds(sid * RPT + k * PCH, PCH)])


def _edge_phase(u_hbm, acc, rowall, colall, rows, semg, sems):
    def _gather(s, slot):
        pltpu.async_copy(u_hbm.at[rowall.at[s]], rows.at[slot], semg[slot])

    def _wait_gather(s, slot):
        pltpu.make_async_copy(u_hbm.at[rowall.at[s]], rows.at[slot],
                              semg[slot]).wait()

    def _scatter(s, slot):
        pltpu.async_copy(rows.at[slot], acc.at[colall.at[s]], sems[slot],
                         add=True)

    def _wait_scatter(s, slot):
        pltpu.make_async_copy(rows.at[slot], acc.at[colall.at[s]],
                              sems[slot]).wait()

    _gather(0, 0)
    _gather(1, 1)

    def body(g, carry):
        for j in range(4):
            s = g * 4 + j
            _wait_gather(s, j)
            _scatter(s, j)

            @pl.when(s >= 2)
            def _():
                _wait_scatter(s - 2, (j - 2) % 4)

            @pl.when(s + 2 < NCH)
            def _():
                _gather(s + 2, (j + 2) % 4)

        return carry

    lax.fori_loop(0, NCH // 4, body, 0)
    _wait_scatter(NCH - 2, 2)
    _wait_scatter(NCH - 1, 3)


def _writeback(acc, out0, out1, cid, sid):
    @pl.when(cid == 0)
    def _():
        pltpu.sync_copy(acc.at[pl.ds(sid * RPT, RPT)],
                        out0.at[pl.ds(sid * RPT, RPT)])

    @pl.when(cid == 1)
    def _():
        pltpu.sync_copy(acc.at[pl.ds(sid * RPT, RPT)],
                        out1.at[pl.ds(sid * RPT, RPT)])


@functools.partial(
    pl.kernel,
    mesh=_mesh,
    compiler_params=_sc_params,
    out_type=[jax.ShapeDtypeStruct((NP, DW), jnp.float32)] * 2,
    scratch_types=[
        pltpu.VMEM((NCH, CH), jnp.int32),
        pltpu.VMEM((CH, DW), jnp.float32),
        pltpu.VMEM((PCH, DW), jnp.float32),
        pltpu.VMEM_SHARED((NP, DW), jnp.float32),
    ],
)
def _sc_deg(edges_hbm, out0, out1, colall, onesbuf, zbuf, dacc):
    cid = lax.axis_index("c")
    sid = lax.axis_index("s")
    wid = sid * NC + cid
    _zero_acc(dacc, zbuf, sid, DW)
    ov = jnp.ones((16,), jnp.float32)

    def fill(r, carry):
        onesbuf[r, pl.ds(0, 16)] = ov
        return carry

    lax.fori_loop(0, CH, fill, 0)
    pltpu.sync_copy(edges_hbm.at[pl.ds(CBASE + wid * NCH, NCH)], colall)
    plsc.subcore_barrier()

    def body(step, carry):
        pltpu.sync_copy(onesbuf, dacc.at[colall.at[step]], add=True)
        return carry

    lax.fori_loop(0, NCH, body, 0)
    plsc.subcore_barrier()
    _writeback(dacc, out0, out1, cid, sid)


@functools.partial(
    pl.kernel,
    mesh=_mesh,
    compiler_params=_sc_params,
    out_type=[jax.ShapeDtypeStruct((NP, C), jnp.float32)] * 2,
    scratch_types=[
        pltpu.VMEM((NCH, CH), jnp.int32),
        pltpu.VMEM((NCH, CH), jnp.int32),
        pltpu.VMEM((4, CH, C), jnp.float32),
        pltpu.VMEM((PCH, C), jnp.float32),
        pltpu.VMEM_SHARED((NP, C), jnp.float32),
        [pltpu.SemaphoreType.DMA] * 4,
        [pltpu.SemaphoreType.DMA] * 4,
    ],
)
def _sc_hop(u_hbm, edges_hbm, out0, out1,
            rowall, colall, rows, zbuf, acc, semg, sems):
    cid = lax.axis_index("c")
    sid = lax.axis_index("s")
    wid = sid * NC + cid
    _zero_acc(acc, zbuf, sid, C)
    pltpu.sync_copy(edges_hbm.at[pl.ds(wid * NCH, NCH)], rowall)
    pltpu.sync_copy(edges_hbm.at[pl.ds(CBASE + wid * NCH, NCH)], colall)
    plsc.subcore_barrier()
    _edge_phase(u_hbm, acc, rowall, colall, rows, semg, sems)
    plsc.subcore_barrier()
    _writeback(acc, out0, out1, cid, sid)


def _tc_prep_body(x_ref, w_ref, d0_ref, d1_ref, z_ref, u0_ref, scl_ref):
    z = lax.dot_general(x_ref[...], w_ref[...], (((1,), (1,)), ((), ())),
                        preferred_element_type=jnp.float32)
    deg = d0_ref[...][:, :1] + d1_ref[...][:, :1] + 2.0
    dis = lax.rsqrt(deg)
    dinv2 = 2.0 / deg
    z_ref[...] = z
    u0_ref[...] = z * dis
    lanes = lax.broadcasted_iota(jnp.int32, (BT, DW), 1)
    scl_ref[...] = jnp.where(lanes == 0, dis, 0.0) + \
        jnp.where(lanes == 1, dinv2, 0.0)


_tc_prep = pl.pallas_call(
    _tc_prep_body,
    grid=(GT,),
    in_specs=[
        pl.BlockSpec((BT, D), lambda i: (i, 0)),
        pl.BlockSpec((C, D), lambda i: (0, 0)),
        pl.BlockSpec((BT, DW), lambda i: (i, 0)),
        pl.BlockSpec((BT, DW), lambda i: (i, 0)),
    ],
    out_specs=[
        pl.BlockSpec((BT, C), lambda i: (i, 0)),
        pl.BlockSpec((BT, C), lambda i: (i, 0)),
        pl.BlockSpec((BT, DW), lambda i: (i, 0)),
    ],
    out_shape=[
        jax.ShapeDtypeStruct((NP, C), jnp.float32),
        jax.ShapeDtypeStruct((NP, C), jnp.float32),
        jax.ShapeDtypeStruct((NP, DW), jnp.float32),
    ],
)


def _tc_comb1_body(p0_ref, p1_ref, scl_ref, h_ref, h1_ref, u1_ref):
    s = scl_ref[...]
    dis = s[:, :1]
    dinv2 = s[:, 1:2]
    h1 = dis * (p0_ref[...] + p1_ref[...]) + dinv2 * h_ref[...]
    h1_ref[...] = h1
    u1_ref[...] = dis * h1


_tc_comb1 = pl.pallas_call(
    _tc_comb1_body,
    grid=(GT,),
    in_specs=[
        pl.BlockSpec((BT, C), lambda i: (i, 0)),
        pl.BlockSpec((BT, C), lambda i: (i, 0)),
        pl.BlockSpec((BT, DW), lambda i: (i, 0)),
        pl.BlockSpec((BT, C), lambda i: (i, 0)),
    ],
    out_specs=[
        pl.BlockSpec((BT, C), lambda i: (i, 0)),
        pl.BlockSpec((BT, C), lambda i: (i, 0)),
    ],
    out_shape=[
        jax.ShapeDtypeStruct((NP, C), jnp.float32),
        jax.ShapeDtypeStruct((NP, C), jnp.float32),
    ],
)


def _tc_comb2_body(p0_ref, p1_ref, scl_ref, h_ref, b_ref, o_ref):
    s = scl_ref[...]
    dis = s[:, :1]
    dinv2 = s[:, 1:2]
    t = dis * (p0_ref[...] + p1_ref[...]) + dinv2 * h_ref[...] + b_ref[...]
    m = jnp.max(t, axis=1, keepdims=True)
    lse = jnp.log(jnp.sum(jnp.exp(t - m), axis=1, keepdims=True)) + m
    o_ref[...] = t - lse


_tc_comb2 = pl.pallas_call(
    _tc_comb2_body,
    grid=(GT,),
    in_specs=[
        pl.BlockSpec((BT, C), lambda i: (i, 0)),
        pl.BlockSpec((BT, C), lambda i: (i, 0)),
        pl.BlockSpec((BT, DW), lambda i: (i, 0)),
        pl.BlockSpec((BT, C), lambda i: (i, 0)),
        pl.BlockSpec((1, C), lambda i: (0, 0)),
    ],
    out_specs=pl.BlockSpec((BT, C), lambda i: (i, 0)),
    out_shape=jax.ShapeDtypeStruct((N, C), jnp.float32),
)


def kernel(x, edge_index, W, b):
    edges = edge_index.reshape(2 * NW * NCH, CH)
    dg0, dg1 = _sc_deg(edges)
    z, u0, scl = _tc_prep(x, W, dg0, dg1)
    p10, p11 = _sc_hop(u0, edges)
    h1, u1 = _tc_comb1(p10, p11, scl, z)
    p20, p21 = _sc_hop(u1, edges)
    return _tc_comb2(p20, p21, scl, h1, b.reshape(1, C))


# speedup vs baseline: 37.3289x; 1.0064x over previous
import functools

import jax
import jax.numpy as jnp
from jax import lax
from jax.experimental import pallas as pl
from jax.experimental.pallas import tpu as pltpu
from jax.experimental.pallas import tpu_sc as plsc

N = 10000
D = 128
C = 64
E = 320000

NC = 2
NS = 16
NW = NC * NS
EPW = E // NW
CH = 125
NCH = EPW // CH
CBASE = NW * NCH
NP = 10240
RPT = NP // NS
DW = 16
PCH = 128

BT = 1000
GT = N // BT

_mesh = plsc.VectorSubcoreMesh(core_axis_name="c", subcore_axis_name="s")
_sc_params = pltpu.CompilerParams(use_tc_tiling_on_sc=False)


def _memset_zero(buf, nrows, ncols):
    zv = jnp.zeros((16,), jnp.float32)

    def body(r, carry):
        for j in range(ncols // 16):
            buf[r, pl.ds(j * 16, 16)] = zv
        return carry

    lax.fori_loop(0, nrows, body, 0)


def _zero_acc(acc, zbuf, sid, width):
    _memset_zero(zbuf, PCH, width)
    for k in range(RPT // PCH):
        pltpu.sync_copy(zbuf, acc.at[pl.ds(sid * RPT + k * PCH, PCH)])


NSLOT = 4
PF = 2


def _edge_phase(u_hbm, acc, rowall, colall, rows, semg, sems):
    def _gather(s, slot):
        pltpu.async_copy(u_hbm.at[rowall.at[s]], rows.at[slot], semg[slot])

    def _wait_gather(s, slot):
        pltpu.make_async_copy(u_hbm.at[rowall.at[s]], rows.at[slot],
                              semg[slot]).wait()

    def _scatter(s, slot):
        pltpu.async_copy(rows.at[slot], acc.at[colall.at[s]], sems[slot],
                         add=True)

    def _wait_scatter(s, slot):
        pltpu.make_async_copy(rows.at[slot], acc.at[colall.at[s]],
                              sems[slot]).wait()

    for s in range(PF):
        _gather(s, s)

    def body(g, carry):
        for j in range(NSLOT):
            s = g * NSLOT + j
            _wait_gather(s, j)
            _scatter(s, j)

            @pl.when(s >= PF)
            def _():
                _wait_scatter(s - PF, (j - PF) % NSLOT)

            @pl.when(s + PF < NCH)
            def _():
                _gather(s + PF, (j + PF) % NSLOT)

        return carry

    lax.fori_loop(0, NCH // NSLOT, body, 0)
    for s in range(NCH - PF, NCH):
        _wait_scatter(s, s % NSLOT)


def _writeback(acc, out0, out1, cid, sid):
    @pl.when(cid == 0)
    def _():
        pltpu.sync_copy(acc.at[pl.ds(sid * RPT, RPT)],
                        out0.at[pl.ds(sid * RPT, RPT)])

    @pl.when(cid == 1)
    def _():
        pltpu.sync_copy(acc.at[pl.ds(sid * RPT, RPT)],
                        out1.at[pl.ds(sid * RPT, RPT)])


@functools.partial(
    pl.kernel,
    mesh=_mesh,
    compiler_params=_sc_params,
    out_type=[jax.ShapeDtypeStruct((NP, DW), jnp.float32)] * 2,
    scratch_types=[
        pltpu.VMEM((NCH, CH), jnp.int32),
        pltpu.VMEM((CH, DW), jnp.float32),
        pltpu.VMEM((PCH, DW), jnp.float32),
        pltpu.VMEM_SHARED((NP, DW), jnp.float32),
    ],
)
def _sc_deg(edges_hbm, out0, out1, colall, onesbuf, zbuf, dacc):
    cid = lax.axis_index("c")
    sid = lax.axis_index("s")
    wid = sid * NC + cid
    _zero_acc(dacc, zbuf, sid, DW)
    ov = jnp.ones((16,), jnp.float32)

    def fill(r, carry):
        onesbuf[r, pl.ds(0, 16)] = ov
        return carry

    lax.fori_loop(0, CH, fill, 0)
    pltpu.sync_copy(edges_hbm.at[pl.ds(CBASE + wid * NCH, NCH)], colall)
    plsc.subcore_barrier()

    def body(step, carry):
        pltpu.sync_copy(onesbuf, dacc.at[colall.at[step]], add=True)
        return carry

    lax.fori_loop(0, NCH, body, 0)
    plsc.subcore_barrier()
    _writeback(dacc, out0, out1, cid, sid)


@functools.partial(
    pl.kernel,
    mesh=_mesh,
    compiler_params=_sc_params,
    out_type=[jax.ShapeDtypeStruct((NP, C), jnp.float32)] * 2,
    scratch_types=[
        pltpu.VMEM((NCH, CH), jnp.int32),
        pltpu.VMEM((NCH, CH), jnp.int32),
        pltpu.VMEM((NSLOT, CH, C), jnp.float32),
        pltpu.VMEM((PCH, C), jnp.float32),
        pltpu.VMEM_SHARED((NP, C), jnp.float32),
        [pltpu.SemaphoreType.DMA] * NSLOT,
        [pltpu.SemaphoreType.DMA] * NSLOT,
    ],
)
def _sc_hop(u_hbm, edges_hbm, out0, out1,
            rowall, colall, rows, zbuf, acc, semg, sems):
    cid = lax.axis_index("c")
    sid = lax.axis_index("s")
    wid = sid * NC + cid
    _zero_acc(acc, zbuf, sid, C)
    pltpu.sync_copy(edges_hbm.at[pl.ds(wid * NCH, NCH)], rowall)
    pltpu.sync_copy(edges_hbm.at[pl.ds(CBASE + wid * NCH, NCH)], colall)
    plsc.subcore_barrier()
    _edge_phase(u_hbm, acc, rowall, colall, rows, semg, sems)
    plsc.subcore_barrier()
    _writeback(acc, out0, out1, cid, sid)


def _tc_matmul_body(x_ref, w_ref, z_ref):
    z_ref[...] = lax.dot_general(x_ref[...], w_ref[...],
                                 (((1,), (1,)), ((), ())),
                                 preferred_element_type=jnp.float32)


_tc_matmul = pl.pallas_call(
    _tc_matmul_body,
    grid=(GT,),
    in_specs=[
        pl.BlockSpec((BT, D), lambda i: (i, 0)),
        pl.BlockSpec((C, D), lambda i: (0, 0)),
    ],
    out_specs=pl.BlockSpec((BT, C), lambda i: (i, 0)),
    out_shape=jax.ShapeDtypeStruct((NP, C), jnp.float32),
)


def _tc_scale_body(z_ref, d0_ref, d1_ref, u0_ref, scl_ref):
    deg = d0_ref[...][:, :1] + d1_ref[...][:, :1] + 2.0
    dis = lax.rsqrt(deg)
    dinv2 = 2.0 / deg
    u0_ref[...] = z_ref[...] * dis
    lanes = lax.broadcasted_iota(jnp.int32, (BT, DW), 1)
    scl_ref[...] = jnp.where(lanes == 0, dis, 0.0) + \
        jnp.where(lanes == 1, dinv2, 0.0)


_tc_scale = pl.pallas_call(
    _tc_scale_body,
    grid=(GT,),
    in_specs=[
        pl.BlockSpec((BT, C), lambda i: (i, 0)),
        pl.BlockSpec((BT, DW), lambda i: (i, 0)),
        pl.BlockSpec((BT, DW), lambda i: (i, 0)),
    ],
    out_specs=[
        pl.BlockSpec((BT, C), lambda i: (i, 0)),
        pl.BlockSpec((BT, DW), lambda i: (i, 0)),
    ],
    out_shape=[
        jax.ShapeDtypeStruct((NP, C), jnp.float32),
        jax.ShapeDtypeStruct((NP, DW), jnp.float32),
    ],
)


def _tc_comb1_body(p0_ref, p1_ref, scl_ref, h_ref, h1_ref, u1_ref):
    s = scl_ref[...]
    dis = s[:, :1]
    dinv2 = s[:, 1:2]
    h1 = dis * (p0_ref[...] + p1_ref[...]) + dinv2 * h_ref[...]
    h1_ref[...] = h1
    u1_ref[...] = dis * h1


_tc_comb1 = pl.pallas_call(
    _tc_comb1_body,
    grid=(GT,),
    in_specs=[
        pl.BlockSpec((BT, C), lambda i: (i, 0)),
        pl.BlockSpec((BT, C), lambda i: (i, 0)),
        pl.BlockSpec((BT, DW), lambda i: (i, 0)),
        pl.BlockSpec((BT, C), lambda i: (i, 0)),
    ],
    out_specs=[
        pl.BlockSpec((BT, C), lambda i: (i, 0)),
        pl.BlockSpec((BT, C), lambda i: (i, 0)),
    ],
    out_shape=[
        jax.ShapeDtypeStruct((NP, C), jnp.float32),
        jax.ShapeDtypeStruct((NP, C), jnp.float32),
    ],
)


def _tc_comb2_body(p0_ref, p1_ref, scl_ref, h_ref, b_ref, o_ref):
    s = scl_ref[...]
    dis = s[:, :1]
    dinv2 = s[:, 1:2]
    t = dis * (p0_ref[...] + p1_ref[...]) + dinv2 * h_ref[...] + b_ref[...]
    m = jnp.max(t, axis=1, keepdims=True)
    lse = jnp.log(jnp.sum(jnp.exp(t - m), axis=1, keepdims=True)) + m
    o_ref[...] = t - lse


_tc_comb2 = pl.pallas_call(
    _tc_comb2_body,
    grid=(GT,),
    in_specs=[
        pl.BlockSpec((BT, C), lambda i: (i, 0)),
        pl.BlockSpec((BT, C), lambda i: (i, 0)),
        pl.BlockSpec((BT, DW), lambda i: (i, 0)),
        pl.BlockSpec((BT, C), lambda i: (i, 0)),
        pl.BlockSpec((1, C), lambda i: (0, 0)),
    ],
    out_specs=pl.BlockSpec((BT, C), lambda i: (i, 0)),
    out_shape=jax.ShapeDtypeStruct((N, C), jnp.float32),
)


def kernel(x, edge_index, W, b):
    edges = edge_index.reshape(2 * NW * NCH, CH)
    dg0, dg1 = _sc_deg(edges)
    z = _tc_matmul(x, W)
    u0, scl = _tc_scale(z, dg0, dg1)
    p10, p11 = _sc_hop(u0, edges)
    h1, u1 = _tc_comb1(p10, p11, scl, z)
    p20, p21 = _sc_hop(u1, edges)
    return _tc_comb2(p20, p21, scl, h1, b.reshape(1, C))


# speedup vs baseline: 38.1140x; 1.0210x over previous
import functools

import jax
import jax.numpy as jnp
from jax import lax
from jax.experimental import pallas as pl
from jax.experimental.pallas import tpu as pltpu
from jax.experimental.pallas import tpu_sc as plsc

N = 10000
D = 128
C = 64
E = 320000

NC = 2
NS = 16
NW = NC * NS
EPW = E // NW
CH = 125
NCH = EPW // CH
CBASE = NW * NCH
NP = 10240
RPT = NP // NS
DW = 16
PCH = 128

BT = 2000
GT = N // BT

_mesh = plsc.VectorSubcoreMesh(core_axis_name="c", subcore_axis_name="s")
_sc_params = pltpu.CompilerParams(use_tc_tiling_on_sc=False)


def _memset_zero(buf, nrows, ncols):
    zv = jnp.zeros((16,), jnp.float32)

    def body(r, carry):
        for j in range(ncols // 16):
            buf[r, pl.ds(j * 16, 16)] = zv
        return carry

    lax.fori_loop(0, nrows, body, 0)


def _zero_acc(acc, zbuf, sid, width):
    _memset_zero(zbuf, PCH, width)
    for k in range(RPT // PCH):
        pltpu.sync_copy(zbuf, acc.at[pl.ds(sid * RPT + k * PCH, PCH)])


NSLOT = 4
PF = 2


def _edge_phase(u_hbm, acc, rowall, colall, rows, semg, sems):
    def _gather(s, slot):
        pltpu.async_copy(u_hbm.at[rowall.at[s]], rows.at[slot], semg[slot])

    def _wait_gather(s, slot):
        pltpu.make_async_copy(u_hbm.at[rowall.at[s]], rows.at[slot],
                              semg[slot]).wait()

    def _scatter(s, slot):
        pltpu.async_copy(rows.at[slot], acc.at[colall.at[s]], sems[slot],
                         add=True)

    def _wait_scatter(s, slot):
        pltpu.make_async_copy(rows.at[slot], acc.at[colall.at[s]],
                              sems[slot]).wait()

    for s in range(PF):
        _gather(s, s)

    def body(g, carry):
        for j in range(NSLOT):
            s = g * NSLOT + j
            _wait_gather(s, j)
            _scatter(s, j)

            @pl.when(s >= PF)
            def _():
                _wait_scatter(s - PF, (j - PF) % NSLOT)

            @pl.when(s + PF < NCH)
            def _():
                _gather(s + PF, (j + PF) % NSLOT)

        return carry

    lax.fori_loop(0, NCH // NSLOT, body, 0)
    for s in range(NCH - PF, NCH):
        _wait_scatter(s, s % NSLOT)


def _writeback(acc, out0, out1, cid, sid):
    @pl.when(cid == 0)
    def _():
        pltpu.sync_copy(acc.at[pl.ds(sid * RPT, RPT)],
                        out0.at[pl.ds(sid * RPT, RPT)])

    @pl.when(cid == 1)
    def _():
        pltpu.sync_copy(acc.at[pl.ds(sid * RPT, RPT)],
                        out1.at[pl.ds(sid * RPT, RPT)])


@functools.partial(
    pl.kernel,
    mesh=_mesh,
    compiler_params=_sc_params,
    out_type=[jax.ShapeDtypeStruct((NP, DW), jnp.float32)] * 2,
    scratch_types=[
        pltpu.VMEM((NCH, CH), jnp.int32),
        pltpu.VMEM((CH, DW), jnp.float32),
        pltpu.VMEM((PCH, DW), jnp.float32),
        pltpu.VMEM_SHARED((NP, DW), jnp.float32),
    ],
)
def _sc_deg(edges_hbm, out0, out1, colall, onesbuf, zbuf, dacc):
    cid = lax.axis_index("c")
    sid = lax.axis_index("s")
    wid = sid * NC + cid
    _zero_acc(dacc, zbuf, sid, DW)
    ov = jnp.ones((16,), jnp.float32)

    def fill(r, carry):
        onesbuf[r, pl.ds(0, 16)] = ov
        return carry

    lax.fori_loop(0, CH, fill, 0)
    pltpu.sync_copy(edges_hbm.at[pl.ds(CBASE + wid * NCH, NCH)], colall)
    plsc.subcore_barrier()

    def body(step, carry):
        pltpu.sync_copy(onesbuf, dacc.at[colall.at[step]], add=True)
        return carry

    lax.fori_loop(0, NCH, body, 0)
    plsc.subcore_barrier()
    _writeback(dacc, out0, out1, cid, sid)


@functools.partial(
    pl.kernel,
    mesh=_mesh,
    compiler_params=_sc_params,
    out_type=[jax.ShapeDtypeStruct((NP, C), jnp.float32)] * 2,
    scratch_types=[
        pltpu.VMEM((NCH, CH), jnp.int32),
        pltpu.VMEM((NCH, CH), jnp.int32),
        pltpu.VMEM((NSLOT, CH, C), jnp.float32),
        pltpu.VMEM((PCH, C), jnp.float32),
        pltpu.VMEM_SHARED((NP, C), jnp.float32),
        [pltpu.SemaphoreType.DMA] * NSLOT,
        [pltpu.SemaphoreType.DMA] * NSLOT,
    ],
)
def _sc_hop(u_hbm, edges_hbm, out0, out1,
            rowall, colall, rows, zbuf, acc, semg, sems):
    cid = lax.axis_index("c")
    sid = lax.axis_index("s")
    wid = sid * NC + cid
    _zero_acc(acc, zbuf, sid, C)
    pltpu.sync_copy(edges_hbm.at[pl.ds(wid * NCH, NCH)], rowall)
    pltpu.sync_copy(edges_hbm.at[pl.ds(CBASE + wid * NCH, NCH)], colall)
    plsc.subcore_barrier()
    _edge_phase(u_hbm, acc, rowall, colall, rows, semg, sems)
    plsc.subcore_barrier()
    _writeback(acc, out0, out1, cid, sid)


def _tc_matmul_body(x_ref, w_ref, z_ref):
    z_ref[...] = lax.dot_general(x_ref[...], w_ref[...],
                                 (((1,), (1,)), ((), ())),
                                 preferred_element_type=jnp.float32)


_tc_matmul = pl.pallas_call(
    _tc_matmul_body,
    grid=(GT,),
    in_specs=[
        pl.BlockSpec((BT, D), lambda i: (i, 0)),
        pl.BlockSpec((C, D), lambda i: (0, 0)),
    ],
    out_specs=pl.BlockSpec((BT, C), lambda i: (i, 0)),
    out_shape=jax.ShapeDtypeStruct((NP, C), jnp.float32),
)


def _tc_scale_body(z_ref, d0_ref, d1_ref, u0_ref, scl_ref):
    deg = d0_ref[...][:, :1] + d1_ref[...][:, :1] + 2.0
    dis = lax.rsqrt(deg)
    dinv2 = 2.0 / deg
    u0_ref[...] = z_ref[...] * dis
    lanes = lax.broadcasted_iota(jnp.int32, (BT, DW), 1)
    scl_ref[...] = jnp.where(lanes == 0, dis, 0.0) + \
        jnp.where(lanes == 1, dinv2, 0.0)


_tc_scale = pl.pallas_call(
    _tc_scale_body,
    grid=(GT,),
    in_specs=[
        pl.BlockSpec((BT, C), lambda i: (i, 0)),
        pl.BlockSpec((BT, DW), lambda i: (i, 0)),
        pl.BlockSpec((BT, DW), lambda i: (i, 0)),
    ],
    out_specs=[
        pl.BlockSpec((BT, C), lambda i: (i, 0)),
        pl.BlockSpec((BT, DW), lambda i: (i, 0)),
    ],
    out_shape=[
        jax.ShapeDtypeStruct((NP, C), jnp.float32),
        jax.ShapeDtypeStruct((NP, DW), jnp.float32),
    ],
)


def _tc_comb1_body(p0_ref, p1_ref, scl_ref, h_ref, h1_ref, u1_ref):
    s = scl_ref[...]
    dis = s[:, :1]
    dinv2 = s[:, 1:2]
    h1 = dis * (p0_ref[...] + p1_ref[...]) + dinv2 * h_ref[...]
    h1_ref[...] = h1
    u1_ref[...] = dis * h1


_tc_comb1 = pl.pallas_call(
    _tc_comb1_body,
    grid=(GT,),
    in_specs=[
        pl.BlockSpec((BT, C), lambda i: (i, 0)),
        pl.BlockSpec((BT, C), lambda i: (i, 0)),
        pl.BlockSpec((BT, DW), lambda i: (i, 0)),
        pl.BlockSpec((BT, C), lambda i: (i, 0)),
    ],
    out_specs=[
        pl.BlockSpec((BT, C), lambda i: (i, 0)),
        pl.BlockSpec((BT, C), lambda i: (i, 0)),
    ],
    out_shape=[
        jax.ShapeDtypeStruct((NP, C), jnp.float32),
        jax.ShapeDtypeStruct((NP, C), jnp.float32),
    ],
)


def _tc_comb2_body(p0_ref, p1_ref, scl_ref, h_ref, b_ref, o_ref):
    s = scl_ref[...]
    dis = s[:, :1]
    dinv2 = s[:, 1:2]
    t = dis * (p0_ref[...] + p1_ref[...]) + dinv2 * h_ref[...] + b_ref[...]
    m = jnp.max(t, axis=1, keepdims=True)
    lse = jnp.log(jnp.sum(jnp.exp(t - m), axis=1, keepdims=True)) + m
    o_ref[...] = t - lse


_tc_comb2 = pl.pallas_call(
    _tc_comb2_body,
    grid=(GT,),
    in_specs=[
        pl.BlockSpec((BT, C), lambda i: (i, 0)),
        pl.BlockSpec((BT, C), lambda i: (i, 0)),
        pl.BlockSpec((BT, DW), lambda i: (i, 0)),
        pl.BlockSpec((BT, C), lambda i: (i, 0)),
        pl.BlockSpec((1, C), lambda i: (0, 0)),
    ],
    out_specs=pl.BlockSpec((BT, C), lambda i: (i, 0)),
    out_shape=jax.ShapeDtypeStruct((N, C), jnp.float32),
)


def kernel(x, edge_index, W, b):
    edges = edge_index.reshape(2 * NW * NCH, CH)
    dg0, dg1 = _sc_deg(edges)
    z = _tc_matmul(x, W)
    u0, scl = _tc_scale(z, dg0, dg1)
    p10, p11 = _sc_hop(u0, edges)
    h1, u1 = _tc_comb1(p10, p11, scl, z)
    p20, p21 = _sc_hop(u1, edges)
    return _tc_comb2(p20, p21, scl, h1, b.reshape(1, C))


# speedup vs baseline: 39.5255x; 1.0370x over previous
import functools

import jax
import jax.numpy as jnp
from jax import lax
from jax.experimental import pallas as pl
from jax.experimental.pallas import tpu as pltpu
from jax.experimental.pallas import tpu_sc as plsc

N = 10000
D = 128
C = 64
E = 320000

NC = 2
NS = 16
NW = NC * NS
EPW = E // NW
CH = 125
NCH = EPW // CH
CBASE = NW * NCH
NP = 10240
RPT = NP // NS
DW = 16
PCH = 128

BT = 2000
GT = N // BT

_mesh = plsc.VectorSubcoreMesh(core_axis_name="c", subcore_axis_name="s")
_sc_params = pltpu.CompilerParams(use_tc_tiling_on_sc=False)


def _memset_zero(buf, nrows, ncols):
    zv = jnp.zeros((16,), jnp.float32)

    def body(r, carry):
        for j in range(ncols // 16):
            buf[r, pl.ds(j * 16, 16)] = zv
        return carry

    lax.fori_loop(0, nrows, body, 0)


def _zero_acc(acc, zbuf, sid, width):
    _memset_zero(zbuf, PCH, width)
    for k in range(RPT // PCH):
        pltpu.sync_copy(zbuf, acc.at[pl.ds(sid * RPT + k * PCH, PCH)])


NSLOT = 4
PF = 2


def _edge_phase(u_hbm, acc, rowall, colall, rows, semg, sems):
    def _gather(s, slot):
        pltpu.async_copy(u_hbm.at[rowall.at[s]], rows.at[slot], semg[slot])

    def _wait_gather(s, slot):
        pltpu.make_async_copy(u_hbm.at[rowall.at[s]], rows.at[slot],
                              semg[slot]).wait()

    def _scatter(s, slot):
        pltpu.async_copy(rows.at[slot], acc.at[colall.at[s]], sems[slot],
                         add=True)

    def _wait_scatter(s, slot):
        pltpu.make_async_copy(rows.at[slot], acc.at[colall.at[s]],
                              sems[slot]).wait()

    for s in range(PF):
        _gather(s, s)

    def body(g, carry):
        for j in range(NSLOT):
            s = g * NSLOT + j
            _wait_gather(s, j)
            _scatter(s, j)

            @pl.when(s >= PF)
            def _():
                _wait_scatter(s - PF, (j - PF) % NSLOT)

            @pl.when(s + PF < NCH)
            def _():
                _gather(s + PF, (j + PF) % NSLOT)

        return carry

    lax.fori_loop(0, NCH // NSLOT, body, 0)
    for s in range(NCH - PF, NCH):
        _wait_scatter(s, s % NSLOT)


def _writeback(acc, out0, out1, cid, sid):
    @pl.when(cid == 0)
    def _():
        pltpu.sync_copy(acc.at[pl.ds(sid * RPT, RPT)],
                        out0.at[pl.ds(sid * RPT, RPT)])

    @pl.when(cid == 1)
    def _():
        pltpu.sync_copy(acc.at[pl.ds(sid * RPT, RPT)],
                        out1.at[pl.ds(sid * RPT, RPT)])


@functools.partial(
    pl.kernel,
    mesh=_mesh,
    compiler_params=_sc_params,
    out_type=[jax.ShapeDtypeStruct((NP, DW), jnp.float32)] * 2,
    scratch_types=[
        pltpu.VMEM((NCH, CH), jnp.int32),
        pltpu.VMEM((CH, DW), jnp.float32),
        pltpu.VMEM((PCH, DW), jnp.float32),
        pltpu.VMEM_SHARED((NP, DW), jnp.float32),
    ],
)
def _sc_deg(edges_hbm, out0, out1, colall, onesbuf, zbuf, dacc):
    cid = lax.axis_index("c")
    sid = lax.axis_index("s")
    wid = sid * NC + cid
    _zero_acc(dacc, zbuf, sid, DW)
    ov = jnp.ones((16,), jnp.float32)

    def fill(r, carry):
        onesbuf[r, pl.ds(0, 16)] = ov
        return carry

    lax.fori_loop(0, CH, fill, 0)
    pltpu.sync_copy(edges_hbm.at[pl.ds(CBASE + wid * NCH, NCH)], colall)
    plsc.subcore_barrier()

    def body(step, carry):
        pltpu.sync_copy(onesbuf, dacc.at[colall.at[step]], add=True)
        return carry

    lax.fori_loop(0, NCH, body, 0)
    plsc.subcore_barrier()
    _writeback(dacc, out0, out1, cid, sid)


@functools.partial(
    pl.kernel,
    mesh=_mesh,
    compiler_params=_sc_params,
    out_type=[jax.ShapeDtypeStruct((NP, C), jnp.float32)] * 2,
    scratch_types=[
        pltpu.VMEM((NCH, CH), jnp.int32),
        pltpu.VMEM((NCH, CH), jnp.int32),
        pltpu.VMEM((NSLOT, CH, C), jnp.float32),
        pltpu.VMEM((PCH, C), jnp.float32),
        pltpu.VMEM_SHARED((NP, C), jnp.float32),
        [pltpu.SemaphoreType.DMA] * NSLOT,
        [pltpu.SemaphoreType.DMA] * NSLOT,
    ],
)
def _sc_hop(u_hbm, edges_hbm, out0, out1,
            rowall, colall, rows, zbuf, acc, semg, sems):
    cid = lax.axis_index("c")
    sid = lax.axis_index("s")
    wid = sid * NC + cid
    _zero_acc(acc, zbuf, sid, C)
    pltpu.sync_copy(edges_hbm.at[pl.ds(wid * NCH, NCH)], rowall)
    pltpu.sync_copy(edges_hbm.at[pl.ds(CBASE + wid * NCH, NCH)], colall)
    plsc.subcore_barrier()
    _edge_phase(u_hbm, acc, rowall, colall, rows, semg, sems)
    plsc.subcore_barrier()
    _writeback(acc, out0, out1, cid, sid)


def _tc_matmul_body(x_ref, w_ref, z_ref):
    z_ref[...] = lax.dot_general(x_ref[...], w_ref[...],
                                 (((1,), (1,)), ((), ())),
                                 preferred_element_type=jnp.float32)


_tc_matmul = pl.pallas_call(
    _tc_matmul_body,
    grid=(GT,),
    in_specs=[
        pl.BlockSpec((BT, D), lambda i: (i, 0)),
        pl.BlockSpec((C, D), lambda i: (0, 0)),
    ],
    out_specs=pl.BlockSpec((BT, C), lambda i: (i, 0)),
    out_shape=jax.ShapeDtypeStruct((NP, C), jnp.float32),
)


def _tc_scale_body(z_ref, d0_ref, d1_ref, u0_ref, scl_ref):
    deg = d0_ref[...][:, :1] + d1_ref[...][:, :1] + 2.0
    dis = lax.rsqrt(deg)
    dinv2 = 2.0 / deg
    u0_ref[...] = z_ref[...] * dis
    lanes = lax.broadcasted_iota(jnp.int32, (BT, DW), 1)
    scl_ref[...] = jnp.where(lanes == 0, dis, 0.0) + \
        jnp.where(lanes == 1, dinv2, 0.0)


_tc_scale = pl.pallas_call(
    _tc_scale_body,
    grid=(GT,),
    in_specs=[
        pl.BlockSpec((BT, C), lambda i: (i, 0)),
        pl.BlockSpec((BT, DW), lambda i: (i, 0)),
        pl.BlockSpec((BT, DW), lambda i: (i, 0)),
    ],
    out_specs=[
        pl.BlockSpec((BT, C), lambda i: (i, 0)),
        pl.BlockSpec((BT, DW), lambda i: (i, 0)),
    ],
    out_shape=[
        jax.ShapeDtypeStruct((NP, C), jnp.float32),
        jax.ShapeDtypeStruct((NP, DW), jnp.float32),
    ],
)


RPW = NP // NW


@functools.partial(
    pl.kernel,
    mesh=_mesh,
    compiler_params=_sc_params,
    out_type=[jax.ShapeDtypeStruct((NP, C), jnp.float32)] * 2,
    scratch_types=[
        pltpu.VMEM((RPW, C), jnp.float32),
        pltpu.VMEM((RPW, C), jnp.float32),
        pltpu.VMEM((RPW, C), jnp.float32),
        pltpu.VMEM((RPW, DW), jnp.float32),
        pltpu.VMEM((RPW, C), jnp.float32),
        pltpu.VMEM((RPW, C), jnp.float32),
    ],
)
def _sc_comb(p0_hbm, p1_hbm, z_hbm, scl_hbm, u1o, h1o,
             p0b, p1b, zb, sclb, u1b, h1b):
    cid = lax.axis_index("c")
    sid = lax.axis_index("s")
    wid = sid * NC + cid
    base = wid * RPW
    pltpu.sync_copy(p0_hbm.at[pl.ds(base, RPW)], p0b)
    pltpu.sync_copy(p1_hbm.at[pl.ds(base, RPW)], p1b)
    pltpu.sync_copy(z_hbm.at[pl.ds(base, RPW)], zb)
    pltpu.sync_copy(scl_hbm.at[pl.ds(base, RPW)], sclb)

    def rowfn(r, carry):
        sv = sclb[r, pl.ds(0, 16)]
        d = sv[0]
        v = sv[1]
        for j in range(C // 16):
            sl = pl.ds(j * 16, 16)
            h = d * (p0b[r, sl] + p1b[r, sl]) + v * zb[r, sl]
            h1b[r, sl] = h
            u1b[r, sl] = d * h
        return carry

    lax.fori_loop(0, RPW, rowfn, 0)
    pltpu.sync_copy(u1b, u1o.at[pl.ds(base, RPW)])
    pltpu.sync_copy(h1b, h1o.at[pl.ds(base, RPW)])


def _tc_comb2_body(p0_ref, p1_ref, scl_ref, h_ref, b_ref, o_ref):
    s = scl_ref[...]
    dis = s[:, :1]
    dinv2 = s[:, 1:2]
    t = dis * (p0_ref[...] + p1_ref[...]) + dinv2 * h_ref[...] + b_ref[...]
    m = jnp.max(t, axis=1, keepdims=True)
    lse = jnp.log(jnp.sum(jnp.exp(t - m), axis=1, keepdims=True)) + m
    o_ref[...] = t - lse


_tc_comb2 = pl.pallas_call(
    _tc_comb2_body,
    grid=(GT,),
    in_specs=[
        pl.BlockSpec((BT, C), lambda i: (i, 0)),
        pl.BlockSpec((BT, C), lambda i: (i, 0)),
        pl.BlockSpec((BT, DW), lambda i: (i, 0)),
        pl.BlockSpec((BT, C), lambda i: (i, 0)),
        pl.BlockSpec((1, C), lambda i: (0, 0)),
    ],
    out_specs=pl.BlockSpec((BT, C), lambda i: (i, 0)),
    out_shape=jax.ShapeDtypeStruct((N, C), jnp.float32),
)


def kernel(x, edge_index, W, b):
    edges = edge_index.reshape(2 * NW * NCH, CH)
    dg0, dg1 = _sc_deg(edges)
    z = _tc_matmul(x, W)
    u0, scl = _tc_scale(z, dg0, dg1)
    p10, p11 = _sc_hop(u0, edges)
    u1, h1 = _sc_comb(p10, p11, z, scl)
    p20, p21 = _sc_hop(u1, edges)
    return _tc_comb2(p20, p21, scl, h1, b.reshape(1, C))
